# Initial kernel scaffold; baseline (speedup 1.0000x reference)
import jax
import jax.numpy as jnp
from jax.experimental import pallas as pl


def kernel(x, edge_index, edge_attr, pos_edge_index, pos_edge_attr, neg_edge_index, neg_edge_attr, params):
    raise NotImplementedError("write your pallas kernel here")


# speedup vs baseline: 16.9681x; 16.9681x over previous
import functools
import math

import jax
import jax.numpy as jnp
from jax import lax
from jax.experimental import pallas as pl
from jax.experimental.pallas import tpu as pltpu
from jax.experimental.pallas import tpu_sc as plsc

N_NODES = 10000
T = 5
F_IN = 100
NH = 100
F_OUT = 20
DPAD = 128
NW = 32
GW = 128
AGG_FCHUNK = 256
FW = 512
AVG_LOG = math.log(17.0)


def _sc_gather_body(table_hbm, idx_hbm, out_hbm, idx_v, rows_v, sem):
    bpw = idx_v.shape[0]
    wid = lax.axis_index("s") * 2 + lax.axis_index("c")
    base = wid * bpw
    pltpu.sync_copy(idx_hbm.at[pl.ds(base, bpw)], idx_v)
    for w in range(bpw // GW):
        pltpu.async_copy(
            table_hbm.at[idx_v.at[pl.ds(w * GW, GW)]], rows_v, sem
        ).wait()
        pltpu.sync_copy(rows_v, out_hbm.at[pl.ds(base + w * GW, GW)])


def _sc_gather(table, idx):
    b = idx.shape[0]
    bpw = b // NW
    mesh = plsc.VectorSubcoreMesh(core_axis_name="c", subcore_axis_name="s")
    kern = pl.kernel(
        _sc_gather_body,
        out_type=jax.ShapeDtypeStruct((b, DPAD), jnp.float32),
        mesh=mesh,
        scratch_types=[
            pltpu.VMEM((bpw,), jnp.int32),
            pltpu.VMEM((GW, DPAD), jnp.float32),
            pltpu.SemaphoreType.DMA,
        ],
    )
    return kern(table, idx)


def _pad_idx(idx, mult):
    b = idx.shape[0]
    pad = (-b) % mult
    if pad:
        fill = jnp.arange(pad, dtype=jnp.int32) % N_NODES
        idx = jnp.concatenate([idx, fill])
    return idx


def _mm_body(x_ref, w_ref, b_ref, o_ref, *, relu):
    acc = jnp.dot(x_ref[...], w_ref[...], preferred_element_type=jnp.float32)
    acc = acc + b_ref[...]
    if relu:
        acc = jnp.maximum(acc, 0.0)
    o_ref[...] = acc


def _mm(x, w, b, relu=False, bm=1000):
    m, k = x.shape
    n = w.shape[1]
    body = functools.partial(_mm_body, relu=relu)
    return pl.pallas_call(
        body,
        grid=(m // bm,),
        in_specs=[
            pl.BlockSpec((bm, k), lambda i: (i, 0)),
            pl.BlockSpec((k, n), lambda i: (0, 0)),
            pl.BlockSpec((1, n), lambda i: (0, 0)),
        ],
        out_specs=pl.BlockSpec((bm, n), lambda i: (i, 0)),
        out_shape=jax.ShapeDtypeStruct((m, n), jnp.float32),
    )(x, w, b.reshape(1, n))


def _mm_res_body(x_ref, w_ref, b_ref, res_ref, o_ref):
    acc = jnp.dot(x_ref[...], w_ref[...], preferred_element_type=jnp.float32)
    o_ref[...] = res_ref[...] + (acc + b_ref[...]) * 0.5


def _mm_res(x, w, b, res, bm=1000):
    m, k = x.shape
    n = w.shape[1]
    return pl.pallas_call(
        _mm_res_body,
        grid=(m // bm,),
        in_specs=[
            pl.BlockSpec((bm, k), lambda i: (i, 0)),
            pl.BlockSpec((k, n), lambda i: (0, 0)),
            pl.BlockSpec((1, n), lambda i: (0, 0)),
            pl.BlockSpec((bm, n), lambda i: (i, 0)),
        ],
        out_specs=pl.BlockSpec((bm, n), lambda i: (i, 0)),
        out_shape=jax.ShapeDtypeStruct((m, n), jnp.float32),
    )(x, w, b.reshape(1, n), res)


def _mm3_body(a_ref, b_ref, c_ref, wa_ref, wb_ref, wc_ref, bias_ref,
              o_ref, *, relu):
    acc = jnp.dot(a_ref[...], wa_ref[...], preferred_element_type=jnp.float32)
    acc += jnp.dot(b_ref[...], wb_ref[...], preferred_element_type=jnp.float32)
    acc += jnp.dot(c_ref[...], wc_ref[...], preferred_element_type=jnp.float32)
    acc = acc + bias_ref[...]
    if relu:
        acc = jnp.maximum(acc, 0.0)
    o_ref[...] = acc


def _mm3(a, b, c, wa, wb, wc, bias, relu=False, bm=1000):
    m = a.shape[0]
    n = wa.shape[1]
    body = functools.partial(_mm3_body, relu=relu)
    return pl.pallas_call(
        body,
        grid=(m // bm,),
        in_specs=[
            pl.BlockSpec((bm, a.shape[1]), lambda i: (i, 0)),
            pl.BlockSpec((bm, b.shape[1]), lambda i: (i, 0)),
            pl.BlockSpec((bm, c.shape[1]), lambda i: (i, 0)),
            pl.BlockSpec((a.shape[1], n), lambda i: (0, 0)),
            pl.BlockSpec((b.shape[1], n), lambda i: (0, 0)),
            pl.BlockSpec((c.shape[1], n), lambda i: (0, 0)),
            pl.BlockSpec((1, n), lambda i: (0, 0)),
        ],
        out_specs=pl.BlockSpec((bm, n), lambda i: (i, 0)),
        out_shape=jax.ShapeDtypeStruct((m, n), jnp.float32),
    )(a, b, c, wa, wb, wc, bias.reshape(1, n))


def _agg_body(dst_ref, msgs_ref, sum_ref, sq_ref, mn_ref, mx_ref, cnt_ref,
              *, eb, with_cnt):
    c = pl.program_id(0)

    @pl.when(c == 0)
    def _init():
        sum_ref[...] = jnp.zeros_like(sum_ref)
        sq_ref[...] = jnp.zeros_like(sq_ref)
        mn_ref[...] = jnp.full_like(mn_ref, jnp.inf)
        mx_ref[...] = jnp.full_like(mx_ref, -jnp.inf)
        if with_cnt:
            cnt_ref[...] = jnp.zeros_like(cnt_ref)

    def step(e, _):
        d = dst_ref[0, 0, e]
        row = msgs_ref[pl.ds(e, 1), :]
        sum_ref[pl.ds(d, 1), :] += row
        sq_ref[pl.ds(d, 1), :] += row * row
        mn_ref[pl.ds(d, 1), :] = jnp.minimum(mn_ref[pl.ds(d, 1), :], row)
        mx_ref[pl.ds(d, 1), :] = jnp.maximum(mx_ref[pl.ds(d, 1), :], row)
        if with_cnt:
            cnt_ref[pl.ds(d, 1), :] += 1.0
        return 0

    lax.fori_loop(0, eb, step, 0)


def _aggregate(msgs, dst3, eb):
    e_tot = msgs.shape[0]
    fw = msgs.shape[1]
    nfc = fw // AGG_FCHUNK
    outs = []
    cnt = None
    for p in range(nfc):
        with_cnt = p == 0
        body = functools.partial(_agg_body, eb=eb, with_cnt=with_cnt)
        agg_spec = pl.BlockSpec((N_NODES, AGG_FCHUNK), lambda c: (0, 0))
        cnt_shape = (N_NODES, 8) if with_cnt else (8, 8)
        res = pl.pallas_call(
            body,
            grid=(e_tot // eb,),
            in_specs=[
                pl.BlockSpec((1, 1, eb), lambda c: (c, 0, 0),
                             memory_space=pltpu.SMEM),
                pl.BlockSpec((eb, AGG_FCHUNK), lambda c, _p=p: (c, _p)),
            ],
            out_specs=[agg_spec] * 4 + [
                pl.BlockSpec(cnt_shape, lambda c: (0, 0))],
            out_shape=[jax.ShapeDtypeStruct((N_NODES, AGG_FCHUNK),
                                            jnp.float32)] * 4 + [
                jax.ShapeDtypeStruct(cnt_shape, jnp.float32)],
        )(dst3, msgs)
        outs.append(res[:4])
        if with_cnt:
            cnt = res[4]
    return outs, cnt


def _post_body(x_ref, sum0_ref, sq0_ref, mn0_ref, mx0_ref,
               sum1_ref, sq1_ref, mn1_ref, mx1_ref, cnt_ref,
               pw_ref, pb_ref, lw_ref, lb_ref, h_ref, st_ref):
    i = pl.program_id(0)
    sum_full = jnp.concatenate([sum0_ref[...], sum1_ref[...]], axis=-1)
    sq_full = jnp.concatenate([sq0_ref[...], sq1_ref[...]], axis=-1)
    mn_full = jnp.concatenate([mn0_ref[...], mn1_ref[...]], axis=-1)
    mx_full = jnp.concatenate([mx0_ref[...], mx1_ref[...]], axis=-1)
    cnt = cnt_ref[:, 0:1]
    cnt_c = jnp.maximum(cnt, 1.0)
    inv = 1.0 / cnt_c
    has = cnt > 0.0
    lg = jnp.log(cnt_c + 1.0)
    amp = lg * (1.0 / AVG_LOG)
    att = AVG_LOG / lg

    feats = []
    for t in range(T):
        s = t * F_IN
        mean = sum_full[:, s:s + F_IN] * inv
        mean2 = sq_full[:, s:s + F_IN] * inv
        std = jnp.sqrt(jnp.maximum(mean2 - mean * mean, 0.0) + 1e-5)
        mnv = jnp.where(has, mn_full[:, s:s + F_IN], 0.0)
        mxv = jnp.where(has, mx_full[:, s:s + F_IN], 0.0)
        agg = jnp.concatenate([mean, mnv, mxv, std], axis=-1)
        feats.append(jnp.concatenate(
            [x_ref[...], agg, agg * amp, agg * att], axis=-1))
    hcat = jnp.concatenate(feats, axis=-1)
    out = jnp.dot(hcat, pw_ref[...], preferred_element_type=jnp.float32)
    out = out + pb_ref[...]
    h = jnp.dot(out, lw_ref[...], preferred_element_type=jnp.float32)
    h = h + lb_ref[...]
    h_ref[...] = h

    @pl.when(i == 0)
    def _():
        st_ref[...] = jnp.zeros_like(st_ref)

    st_ref[0:1, 0:NH] += jnp.sum(h, axis=0, keepdims=True)
    st_ref[1:2, 0:NH] += jnp.sum(h * h, axis=0, keepdims=True)


def _post(x, aggs, cnt, pw, pb, lw, lb, bm=400):
    agg_args = list(aggs[0]) + list(aggs[1])
    return pl.pallas_call(
        _post_body,
        grid=(N_NODES // bm,),
        in_specs=[
            pl.BlockSpec((bm, NH), lambda i: (i, 0)),
        ] + [
            pl.BlockSpec((bm, AGG_FCHUNK), lambda i: (i, 0))
            for _ in range(8)
        ] + [
            pl.BlockSpec((bm, 8), lambda i: (i, 0)),
            pl.BlockSpec((T * 1300, T * F_OUT), lambda i: (0, 0)),
            pl.BlockSpec((1, T * F_OUT), lambda i: (0, 0)),
            pl.BlockSpec((NH, NH), lambda i: (0, 0)),
            pl.BlockSpec((1, NH), lambda i: (0, 0)),
        ],
        out_specs=[
            pl.BlockSpec((bm, NH), lambda i: (i, 0)),
            pl.BlockSpec((8, 128), lambda i: (0, 0)),
        ],
        out_shape=[
            jax.ShapeDtypeStruct((N_NODES, NH), jnp.float32),
            jax.ShapeDtypeStruct((8, 128), jnp.float32),
        ],
    )(x, *agg_args, cnt, pw, pb.reshape(1, T * F_OUT), lw,
      lb.reshape(1, NH))


def _bn_res_body(x_ref, h_ref, st_ref, g_ref, b_ref, o_ref, op_ref):
    mean = st_ref[0:1, 0:NH] * (1.0 / N_NODES)
    var = st_ref[1:2, 0:NH] * (1.0 / N_NODES) - mean * mean
    rstd = lax.rsqrt(var + 1e-5)
    hn = g_ref[...] * (h_ref[...] - mean) * rstd + b_ref[...]
    xn = (x_ref[...] + jnp.maximum(hn, 0.0)) * 0.5
    o_ref[...] = xn
    op_ref[...] = jnp.concatenate(
        [xn, jnp.zeros((xn.shape[0], DPAD - NH), jnp.float32)], axis=-1)


def _bn_res(x, h, st, g, b, bm=1000):
    return pl.pallas_call(
        _bn_res_body,
        grid=(N_NODES // bm,),
        in_specs=[
            pl.BlockSpec((bm, NH), lambda i: (i, 0)),
            pl.BlockSpec((bm, NH), lambda i: (i, 0)),
            pl.BlockSpec((8, 128), lambda i: (0, 0)),
            pl.BlockSpec((1, NH), lambda i: (0, 0)),
            pl.BlockSpec((1, NH), lambda i: (0, 0)),
        ],
        out_specs=[
            pl.BlockSpec((bm, NH), lambda i: (i, 0)),
            pl.BlockSpec((bm, DPAD), lambda i: (i, 0)),
        ],
        out_shape=[
            jax.ShapeDtypeStruct((N_NODES, NH), jnp.float32),
            jax.ShapeDtypeStruct((N_NODES, DPAD), jnp.float32),
        ],
    )(x, h, st, g.reshape(1, NH), b.reshape(1, NH))


def _head_body(a_ref, b_ref, c_ref, w1a_ref, w1b_ref, w1c_ref, b1_ref,
               w2_ref, b2_ref, o_ref):
    z = jnp.dot(a_ref[...], w1a_ref[...], preferred_element_type=jnp.float32)
    z += jnp.dot(b_ref[...], w1b_ref[...], preferred_element_type=jnp.float32)
    z += jnp.dot(c_ref[...], w1c_ref[...], preferred_element_type=jnp.float32)
    z = jnp.maximum(z + b1_ref[...], 0.0)
    y = jnp.dot(z, w2_ref[...], preferred_element_type=jnp.float32)
    o_ref[...] = jax.nn.sigmoid(y + b2_ref[...])


def _head(xs, xd, ea, w1a, w1b, w1c, b1, w2, b2, bm=1000):
    m = xs.shape[0]
    w2p = jnp.concatenate([w2, jnp.zeros((NH, 127), jnp.float32)], axis=1)
    b2p = jnp.concatenate([b2, jnp.zeros((127,), jnp.float32)]).reshape(1, 128)
    return pl.pallas_call(
        _head_body,
        grid=(m // bm,),
        in_specs=[
            pl.BlockSpec((bm, DPAD), lambda i: (i, 0)),
            pl.BlockSpec((bm, DPAD), lambda i: (i, 0)),
            pl.BlockSpec((bm, NH), lambda i: (i, 0)),
            pl.BlockSpec((DPAD, NH), lambda i: (0, 0)),
            pl.BlockSpec((DPAD, NH), lambda i: (0, 0)),
            pl.BlockSpec((NH, NH), lambda i: (0, 0)),
            pl.BlockSpec((1, NH), lambda i: (0, 0)),
            pl.BlockSpec((NH, 128), lambda i: (0, 0)),
            pl.BlockSpec((1, 128), lambda i: (0, 0)),
        ],
        out_specs=pl.BlockSpec((bm, 128), lambda i: (i, 0)),
        out_shape=jax.ShapeDtypeStruct((m, 128), jnp.float32),
    )(xs, xd, ea, w1a, w1b, w1c, b1.reshape(1, NH), w2p, b2p)


def _padw(w):
    return jnp.concatenate([w, jnp.zeros((DPAD - NH, w.shape[1]), w.dtype)])


def kernel(x, edge_index, edge_attr, pos_edge_index, pos_edge_attr,
           neg_edge_index, neg_edge_attr, params):
    e = edge_index.shape[1]
    e_lp = pos_edge_index.shape[1]
    src, dst = edge_index[0], edge_index[1]

    x0 = _mm(x, params["node_emb"]["w"], params["node_emb"]["b"])
    eattr = _mm(edge_attr, params["edge_emb"]["w"], params["edge_emb"]["b"])
    pos_ea = _mm(pos_edge_attr, params["edge_emb"]["w"],
                 params["edge_emb"]["b"])
    neg_ea = _mm(neg_edge_attr, params["edge_emb"]["w"],
                 params["edge_emb"]["b"])

    idx_layer = _pad_idx(jnp.concatenate([dst, src]), NW * GW)
    idx_heads = _pad_idx(
        jnp.concatenate([pos_edge_index[0], pos_edge_index[1],
                         neg_edge_index[0], neg_edge_index[1]]), NW * GW)

    eb = 2000
    dst3 = dst.reshape(e // eb, 1, eb)

    xt = x0
    xt_pad = jnp.concatenate(
        [xt, jnp.zeros((N_NODES, DPAD - NH), jnp.float32)], axis=-1)
    g = None
    for lp in params["layers"]:
        if g is None:
            g = _sc_gather(xt_pad, idx_layer)
        xd_g, xs_g = g[:e], g[e:2 * e]

        wpre = jnp.concatenate([p["w"] for p in lp["pre"]], axis=1)
        bpre = jnp.concatenate([p["b"] for p in lp["pre"]])
        wpre = jnp.pad(wpre, ((0, 0), (0, FW - T * F_IN)))
        bpre = jnp.pad(bpre, (0, FW - T * F_IN))
        wd, ws, we = wpre[:NH], wpre[NH:2 * NH], wpre[2 * NH:]
        wee = lp["edge_enc"]["w"] @ we
        bee = lp["edge_enc"]["b"] @ we + bpre
        msgs = _mm3(xd_g, xs_g, eattr, _padw(wd), _padw(ws), wee, bee)

        aggs, cnt = _aggregate(msgs, dst3, eb)

        pwb = jnp.zeros((T * 1300, T * F_OUT), jnp.float32)
        for t in range(T):
            pwb = pwb.at[t * 1300:(t + 1) * 1300,
                         t * F_OUT:(t + 1) * F_OUT].set(lp["post"][t]["w"])
        pbb = jnp.concatenate([p["b"] for p in lp["post"]])

        h, st = _post(xt, aggs, cnt, pwb, pbb,
                      lp["lin"]["w"], lp["lin"]["b"])
        xt, xt_pad = _bn_res(xt, h, st, lp["bn_g"], lp["bn_b"])

        g = _sc_gather(xt_pad, idx_layer)
        xd2, xs2 = g[:e], g[e:2 * e]
        w1 = lp["emlp1"]["w"]
        z = _mm3(xs2, xd2, eattr, _padw(w1[:NH]), _padw(w1[NH:2 * NH]),
                 w1[2 * NH:], lp["emlp1"]["b"], relu=True)
        eattr = _mm_res(z, lp["emlp2"]["w"], lp["emlp2"]["b"], eattr)

    gh = _sc_gather(xt_pad, idx_heads)
    ps, pd = gh[:e_lp], gh[e_lp:2 * e_lp]
    ns, nd = gh[2 * e_lp:3 * e_lp], gh[3 * e_lp:4 * e_lp]
    d1 = params["dec1"]["w"]
    pos_out = _head(ps, pd, pos_ea, _padw(d1[:NH]), _padw(d1[NH:2 * NH]),
                    d1[2 * NH:], params["dec1"]["b"],
                    params["dec2"]["w"], params["dec2"]["b"])
    neg_out = _head(ns, nd, neg_ea, _padw(d1[:NH]), _padw(d1[NH:2 * NH]),
                    d1[2 * NH:], params["dec1"]["b"],
                    params["dec2"]["w"], params["dec2"]["b"])
    return pos_out[:, 0], neg_out[:, 0]


# speedup vs baseline: 28.3255x; 1.6693x over previous
import functools
import math

import jax
import jax.numpy as jnp
from jax import lax
from jax.experimental import pallas as pl
from jax.experimental.pallas import tpu as pltpu
from jax.experimental.pallas import tpu_sc as plsc

N_NODES = 10000
T = 5
F_IN = 100
NH = 100
F_OUT = 20
DPAD = 128
NW = 32
GW = 128
AGG_FCHUNK = 256
FW = 512
AVG_LOG = math.log(17.0)


def _sc_gather_body(table_hbm, idx_hbm, out_hbm, idx_v, rows_v, sem):
    bpw = idx_v.shape[0]
    wid = lax.axis_index("s") * 2 + lax.axis_index("c")
    base = wid * bpw
    pltpu.sync_copy(idx_hbm.at[pl.ds(base, bpw)], idx_v)
    for w in range(bpw // GW):
        pltpu.async_copy(
            table_hbm.at[idx_v.at[pl.ds(w * GW, GW)]], rows_v, sem
        ).wait()
        pltpu.sync_copy(rows_v, out_hbm.at[pl.ds(base + w * GW, GW)])


def _sc_gather(table, idx):
    b = idx.shape[0]
    bpw = b // NW
    mesh = plsc.VectorSubcoreMesh(core_axis_name="c", subcore_axis_name="s")
    kern = pl.kernel(
        _sc_gather_body,
        out_type=jax.ShapeDtypeStruct((b, DPAD), jnp.float32),
        mesh=mesh,
        scratch_types=[
            pltpu.VMEM((bpw,), jnp.int32),
            pltpu.VMEM((GW, DPAD), jnp.float32),
            pltpu.SemaphoreType.DMA,
        ],
    )
    return kern(table, idx)


SEG_WIN = 80
SEG_NWIN = 125
HALF = 5120
DUMP = 8


def _sc_segsum_body(m_hbm, m2_hbm, idx_hbm, out_hbm,
                    zero_v, idx_a, idx_b, rows_a, rows_b,
                    acc_sh, sems):
    c = lax.axis_index("c")
    s = lax.axis_index("s")
    e_tot = SEG_WIN * SEG_NWIN * 16
    nrows = HALF // 16
    row0 = pl.multiple_of(s * nrows, 8)

    @pl.loop(0, nrows)
    def _(r):
        for k in range(8):
            zero_v[r, pl.ds(k * 16, 16)] = jnp.zeros((16,), jnp.float32)

    for r in range(8):
        src = m_hbm if r < 4 else m2_hbm
        q_out = r
        col0 = (r % 4) * 128

        pltpu.sync_copy(zero_v, acc_sh.at[pl.ds(row0, nrows)])
        plsc.subcore_barrier()

        bufs = ((idx_a, rows_a, sems.at[0], sems.at[1]),
                (idx_b, rows_b, sems.at[2], sems.at[3]))

        def issue(w, bi):
            idx_v, rows_v, sem_i, sem_r = bufs[bi]
            j = s * SEG_NWIN + w
            roff = pl.multiple_of(j * SEG_WIN, 8)
            ioff = pl.multiple_of(c * e_tot + j * SEG_WIN, 8)
            h1 = pltpu.async_copy(idx_hbm.at[pl.ds(ioff, SEG_WIN)], idx_v,
                                  sem_i)
            h2 = pltpu.async_copy(
                src.at[pl.ds(roff, SEG_WIN), pl.ds(col0, 128)],
                rows_v, sem_r)
            return h1, h2

        def pair(w):
            hs = (issue(w, 0), issue(w + 1, 1))
            for b in range(2):
                idx_v, rows_v, _, _ = bufs[b]
                hs[b][0].wait()
                hs[b][1].wait()
                pltpu.sync_copy(rows_v, acc_sh.at[idx_v], add=True)

        @pl.loop(0, SEG_NWIN - 1, step=2)
        def _(w):
            pair(w)

        hl = issue(SEG_NWIN - 1, 0)
        hl[0].wait()
        hl[1].wait()
        pltpu.sync_copy(rows_a, acc_sh.at[idx_a], add=True)

        plsc.subcore_barrier()
        pltpu.sync_copy(acc_sh.at[pl.ds(row0, nrows)],
                        out_hbm.at[q_out].at[c].at[pl.ds(row0, nrows)])
        plsc.subcore_barrier()


def _sc_segsum(msgs, msgs2, idx_flat):
    mesh = plsc.VectorSubcoreMesh(core_axis_name="c", subcore_axis_name="s")
    kern = pl.kernel(
        _sc_segsum_body,
        out_type=jax.ShapeDtypeStruct((8, 2, HALF, 128), jnp.float32),
        mesh=mesh,
        scratch_types=[
            pltpu.VMEM((HALF // 16, 128), jnp.float32),
            pltpu.VMEM((SEG_WIN,), jnp.int32),
            pltpu.VMEM((SEG_WIN,), jnp.int32),
            pltpu.VMEM((SEG_WIN, 128), jnp.float32),
            pltpu.VMEM((SEG_WIN, 128), jnp.float32),
            pltpu.VMEM_SHARED((HALF + DUMP, 128), jnp.float32),
            pltpu.SemaphoreType.DMA((4,)),
        ],
    )
    return kern(msgs, msgs2, idx_flat)


def _pad_idx(idx, mult):
    b = idx.shape[0]
    pad = (-b) % mult
    if pad:
        fill = jnp.arange(pad, dtype=jnp.int32) % N_NODES
        idx = jnp.concatenate([idx, fill])
    return idx


def _mm_body(x_ref, w_ref, b_ref, o_ref, *, relu):
    acc = jnp.dot(x_ref[...], w_ref[...], preferred_element_type=jnp.float32)
    acc = acc + b_ref[...]
    if relu:
        acc = jnp.maximum(acc, 0.0)
    o_ref[...] = acc


def _mm(x, w, b, relu=False, bm=1000):
    m, k = x.shape
    n = w.shape[1]
    body = functools.partial(_mm_body, relu=relu)
    return pl.pallas_call(
        body,
        grid=(m // bm,),
        in_specs=[
            pl.BlockSpec((bm, k), lambda i: (i, 0)),
            pl.BlockSpec((k, n), lambda i: (0, 0)),
            pl.BlockSpec((1, n), lambda i: (0, 0)),
        ],
        out_specs=pl.BlockSpec((bm, n), lambda i: (i, 0)),
        out_shape=jax.ShapeDtypeStruct((m, n), jnp.float32),
    )(x, w, b.reshape(1, n))


def _mm_res_body(x_ref, w_ref, b_ref, res_ref, o_ref):
    acc = jnp.dot(x_ref[...], w_ref[...], preferred_element_type=jnp.float32)
    o_ref[...] = res_ref[...] + (acc + b_ref[...]) * 0.5


def _mm_res(x, w, b, res, bm=1000):
    m, k = x.shape
    n = w.shape[1]
    return pl.pallas_call(
        _mm_res_body,
        grid=(m // bm,),
        in_specs=[
            pl.BlockSpec((bm, k), lambda i: (i, 0)),
            pl.BlockSpec((k, n), lambda i: (0, 0)),
            pl.BlockSpec((1, n), lambda i: (0, 0)),
            pl.BlockSpec((bm, n), lambda i: (i, 0)),
        ],
        out_specs=pl.BlockSpec((bm, n), lambda i: (i, 0)),
        out_shape=jax.ShapeDtypeStruct((m, n), jnp.float32),
    )(x, w, b.reshape(1, n), res)


def _mm3_body(a_ref, b_ref, c_ref, wa_ref, wb_ref, wc_ref, bias_ref,
              o_ref, *, relu):
    acc = jnp.dot(a_ref[...], wa_ref[...], preferred_element_type=jnp.float32)
    acc += jnp.dot(b_ref[...], wb_ref[...], preferred_element_type=jnp.float32)
    acc += jnp.dot(c_ref[...], wc_ref[...], preferred_element_type=jnp.float32)
    acc = acc + bias_ref[...]
    if relu:
        acc = jnp.maximum(acc, 0.0)
    o_ref[...] = acc


def _mm3_sq_body(a_ref, b_ref, c_ref, wa_ref, wb_ref, wc_ref, bias_ref,
                 o_ref, o2_ref):
    acc = jnp.dot(a_ref[...], wa_ref[...], preferred_element_type=jnp.float32)
    acc += jnp.dot(b_ref[...], wb_ref[...], preferred_element_type=jnp.float32)
    acc += jnp.dot(c_ref[...], wc_ref[...], preferred_element_type=jnp.float32)
    acc = acc + bias_ref[...]
    o_ref[...] = acc
    o2_ref[...] = acc * acc


def _mm3_sq(a, b, c, wa, wb, wc, bias, bm=1000):
    m = a.shape[0]
    n = wa.shape[1]
    return pl.pallas_call(
        _mm3_sq_body,
        grid=(m // bm,),
        in_specs=[
            pl.BlockSpec((bm, a.shape[1]), lambda i: (i, 0)),
            pl.BlockSpec((bm, b.shape[1]), lambda i: (i, 0)),
            pl.BlockSpec((bm, c.shape[1]), lambda i: (i, 0)),
            pl.BlockSpec((a.shape[1], n), lambda i: (0, 0)),
            pl.BlockSpec((b.shape[1], n), lambda i: (0, 0)),
            pl.BlockSpec((c.shape[1], n), lambda i: (0, 0)),
            pl.BlockSpec((1, n), lambda i: (0, 0)),
        ],
        out_specs=[pl.BlockSpec((bm, n), lambda i: (i, 0))] * 2,
        out_shape=[jax.ShapeDtypeStruct((m, n), jnp.float32)] * 2,
    )(a, b, c, wa, wb, wc, bias.reshape(1, n))


def _mm3(a, b, c, wa, wb, wc, bias, relu=False, bm=1000):
    m = a.shape[0]
    n = wa.shape[1]
    body = functools.partial(_mm3_body, relu=relu)
    return pl.pallas_call(
        body,
        grid=(m // bm,),
        in_specs=[
            pl.BlockSpec((bm, a.shape[1]), lambda i: (i, 0)),
            pl.BlockSpec((bm, b.shape[1]), lambda i: (i, 0)),
            pl.BlockSpec((bm, c.shape[1]), lambda i: (i, 0)),
            pl.BlockSpec((a.shape[1], n), lambda i: (0, 0)),
            pl.BlockSpec((b.shape[1], n), lambda i: (0, 0)),
            pl.BlockSpec((c.shape[1], n), lambda i: (0, 0)),
            pl.BlockSpec((1, n), lambda i: (0, 0)),
        ],
        out_specs=pl.BlockSpec((bm, n), lambda i: (i, 0)),
        out_shape=jax.ShapeDtypeStruct((m, n), jnp.float32),
    )(a, b, c, wa, wb, wc, bias.reshape(1, n))


def _minmax_body(dst_ref, msgs_ref, mn_ref, mx_ref, cnt_ref, *, eb):
    c = pl.program_id(0)

    @pl.when(c == 0)
    def _init():
        mn_ref[...] = jnp.full_like(mn_ref, jnp.inf)
        mx_ref[...] = jnp.full_like(mx_ref, -jnp.inf)
        cnt_ref[...] = jnp.zeros_like(cnt_ref)

    def step(e, _):
        d = dst_ref[0, 0, e]
        row = msgs_ref[pl.ds(e, 1), :]
        mn_ref[pl.ds(d, 1), :] = jnp.minimum(mn_ref[pl.ds(d, 1), :], row)
        mx_ref[pl.ds(d, 1), :] = jnp.maximum(mx_ref[pl.ds(d, 1), :], row)
        cnt_ref[pl.ds(d, 1), :] += 1.0
        return 0

    lax.fori_loop(0, eb, step, 0)


def _minmax_cnt(msgs, dst3, eb):
    e_tot = msgs.shape[0]
    fw = msgs.shape[1]
    body = functools.partial(_minmax_body, eb=eb)
    return pl.pallas_call(
        body,
        grid=(e_tot // eb,),
        in_specs=[
            pl.BlockSpec((1, 1, eb), lambda c: (c, 0, 0),
                         memory_space=pltpu.SMEM),
            pl.BlockSpec((eb, fw), lambda c: (c, 0)),
        ],
        out_specs=[
            pl.BlockSpec((N_NODES, fw), lambda c: (0, 0)),
            pl.BlockSpec((N_NODES, fw), lambda c: (0, 0)),
            pl.BlockSpec((N_NODES, 8), lambda c: (0, 0)),
        ],
        out_shape=[
            jax.ShapeDtypeStruct((N_NODES, fw), jnp.float32),
            jax.ShapeDtypeStruct((N_NODES, fw), jnp.float32),
            jax.ShapeDtypeStruct((N_NODES, 8), jnp.float32),
        ],
    )(dst3, msgs)


def _agg_body(dst_ref, msgs_ref, sum_ref, sq_ref, mn_ref, mx_ref, cnt_ref,
              *, eb, with_cnt):
    c = pl.program_id(0)

    @pl.when(c == 0)
    def _init():
        sum_ref[...] = jnp.zeros_like(sum_ref)
        sq_ref[...] = jnp.zeros_like(sq_ref)
        mn_ref[...] = jnp.full_like(mn_ref, jnp.inf)
        mx_ref[...] = jnp.full_like(mx_ref, -jnp.inf)
        if with_cnt:
            cnt_ref[...] = jnp.zeros_like(cnt_ref)

    def step(e, _):
        d = dst_ref[0, 0, e]
        row = msgs_ref[pl.ds(e, 1), :]
        sum_ref[pl.ds(d, 1), :] += row
        sq_ref[pl.ds(d, 1), :] += row * row
        mn_ref[pl.ds(d, 1), :] = jnp.minimum(mn_ref[pl.ds(d, 1), :], row)
        mx_ref[pl.ds(d, 1), :] = jnp.maximum(mx_ref[pl.ds(d, 1), :], row)
        if with_cnt:
            cnt_ref[pl.ds(d, 1), :] += 1.0
        return 0

    lax.fori_loop(0, eb, step, 0)


def _aggregate(msgs, dst3, eb):
    e_tot = msgs.shape[0]
    fw = msgs.shape[1]
    nfc = fw // AGG_FCHUNK
    outs = []
    cnt = None
    for p in range(nfc):
        with_cnt = p == 0
        body = functools.partial(_agg_body, eb=eb, with_cnt=with_cnt)
        agg_spec = pl.BlockSpec((N_NODES, AGG_FCHUNK), lambda c: (0, 0))
        cnt_shape = (N_NODES, 8) if with_cnt else (8, 8)
        res = pl.pallas_call(
            body,
            grid=(e_tot // eb,),
            in_specs=[
                pl.BlockSpec((1, 1, eb), lambda c: (c, 0, 0),
                             memory_space=pltpu.SMEM),
                pl.BlockSpec((eb, AGG_FCHUNK), lambda c, _p=p: (c, _p)),
            ],
            out_specs=[agg_spec] * 4 + [
                pl.BlockSpec(cnt_shape, lambda c: (0, 0))],
            out_shape=[jax.ShapeDtypeStruct((N_NODES, AGG_FCHUNK),
                                            jnp.float32)] * 4 + [
                jax.ShapeDtypeStruct(cnt_shape, jnp.float32)],
        )(dst3, msgs)
        outs.append(res[:4])
        if with_cnt:
            cnt = res[4]
    return outs, cnt


def _post_body(x_ref, sum_ref, sq_ref, mn_ref, mx_ref, cnt_ref,
               pw_ref, pb_ref, lw_ref, lb_ref, h_ref, st_ref):
    i = pl.program_id(0)
    sum_full = sum_ref[...]
    sq_full = sq_ref[...]
    mn_full = mn_ref[...]
    mx_full = mx_ref[...]
    cnt = cnt_ref[:, 0:1]
    cnt_c = jnp.maximum(cnt, 1.0)
    inv = 1.0 / cnt_c
    has = cnt > 0.0
    lg = jnp.log(cnt_c + 1.0)
    amp = lg * (1.0 / AVG_LOG)
    att = AVG_LOG / lg

    feats = []
    for t in range(T):
        s = t * F_IN
        mean = sum_full[:, s:s + F_IN] * inv
        mean2 = sq_full[:, s:s + F_IN] * inv
        std = jnp.sqrt(jnp.maximum(mean2 - mean * mean, 0.0) + 1e-5)
        mnv = jnp.where(has, mn_full[:, s:s + F_IN], 0.0)
        mxv = jnp.where(has, mx_full[:, s:s + F_IN], 0.0)
        agg = jnp.concatenate([mean, mnv, mxv, std], axis=-1)
        feats.append(jnp.concatenate(
            [x_ref[...], agg, agg * amp, agg * att], axis=-1))
    hcat = jnp.concatenate(feats, axis=-1)
    out = jnp.dot(hcat, pw_ref[...], preferred_element_type=jnp.float32)
    out = out + pb_ref[...]
    h = jnp.dot(out, lw_ref[...], preferred_element_type=jnp.float32)
    h = h + lb_ref[...]
    h_ref[...] = h

    @pl.when(i == 0)
    def _():
        st_ref[...] = jnp.zeros_like(st_ref)

    st_ref[0:1, 0:NH] += jnp.sum(h, axis=0, keepdims=True)
    st_ref[1:2, 0:NH] += jnp.sum(h * h, axis=0, keepdims=True)


def _post(x, sum_full, sq_full, mn_a, mx_a, cnt, pw, pb, lw, lb, bm=400):
    return pl.pallas_call(
        _post_body,
        grid=(N_NODES // bm,),
        in_specs=[
            pl.BlockSpec((bm, NH), lambda i: (i, 0)),
        ] + [
            pl.BlockSpec((bm, FW), lambda i: (i, 0))
            for _ in range(4)
        ] + [
            pl.BlockSpec((bm, 8), lambda i: (i, 0)),
            pl.BlockSpec((T * 1300, T * F_OUT), lambda i: (0, 0)),
            pl.BlockSpec((1, T * F_OUT), lambda i: (0, 0)),
            pl.BlockSpec((NH, NH), lambda i: (0, 0)),
            pl.BlockSpec((1, NH), lambda i: (0, 0)),
        ],
        out_specs=[
            pl.BlockSpec((bm, NH), lambda i: (i, 0)),
            pl.BlockSpec((8, 128), lambda i: (0, 0)),
        ],
        out_shape=[
            jax.ShapeDtypeStruct((N_NODES, NH), jnp.float32),
            jax.ShapeDtypeStruct((8, 128), jnp.float32),
        ],
    )(x, sum_full, sq_full, mn_a, mx_a, cnt, pw, pb.reshape(1, T * F_OUT), lw,
      lb.reshape(1, NH))


def _bn_res_body(x_ref, h_ref, st_ref, g_ref, b_ref, o_ref, op_ref):
    mean = st_ref[0:1, 0:NH] * (1.0 / N_NODES)
    var = st_ref[1:2, 0:NH] * (1.0 / N_NODES) - mean * mean
    rstd = lax.rsqrt(var + 1e-5)
    hn = g_ref[...] * (h_ref[...] - mean) * rstd + b_ref[...]
    xn = (x_ref[...] + jnp.maximum(hn, 0.0)) * 0.5
    o_ref[...] = xn
    op_ref[...] = jnp.concatenate(
        [xn, jnp.zeros((xn.shape[0], DPAD - NH), jnp.float32)], axis=-1)


def _bn_res(x, h, st, g, b, bm=1000):
    return pl.pallas_call(
        _bn_res_body,
        grid=(N_NODES // bm,),
        in_specs=[
            pl.BlockSpec((bm, NH), lambda i: (i, 0)),
            pl.BlockSpec((bm, NH), lambda i: (i, 0)),
            pl.BlockSpec((8, 128), lambda i: (0, 0)),
            pl.BlockSpec((1, NH), lambda i: (0, 0)),
            pl.BlockSpec((1, NH), lambda i: (0, 0)),
        ],
        out_specs=[
            pl.BlockSpec((bm, NH), lambda i: (i, 0)),
            pl.BlockSpec((bm, DPAD), lambda i: (i, 0)),
        ],
        out_shape=[
            jax.ShapeDtypeStruct((N_NODES, NH), jnp.float32),
            jax.ShapeDtypeStruct((N_NODES, DPAD), jnp.float32),
        ],
    )(x, h, st, g.reshape(1, NH), b.reshape(1, NH))


def _head_body(a_ref, b_ref, c_ref, w1a_ref, w1b_ref, w1c_ref, b1_ref,
               w2_ref, b2_ref, o_ref):
    z = jnp.dot(a_ref[...], w1a_ref[...], preferred_element_type=jnp.float32)
    z += jnp.dot(b_ref[...], w1b_ref[...], preferred_element_type=jnp.float32)
    z += jnp.dot(c_ref[...], w1c_ref[...], preferred_element_type=jnp.float32)
    z = jnp.maximum(z + b1_ref[...], 0.0)
    y = jnp.dot(z, w2_ref[...], preferred_element_type=jnp.float32)
    o_ref[...] = jax.nn.sigmoid(y + b2_ref[...])


def _head(xs, xd, ea, w1a, w1b, w1c, b1, w2, b2, bm=1000):
    m = xs.shape[0]
    w2p = jnp.concatenate([w2, jnp.zeros((NH, 127), jnp.float32)], axis=1)
    b2p = jnp.concatenate([b2, jnp.zeros((127,), jnp.float32)]).reshape(1, 128)
    return pl.pallas_call(
        _head_body,
        grid=(m // bm,),
        in_specs=[
            pl.BlockSpec((bm, DPAD), lambda i: (i, 0)),
            pl.BlockSpec((bm, DPAD), lambda i: (i, 0)),
            pl.BlockSpec((bm, NH), lambda i: (i, 0)),
            pl.BlockSpec((DPAD, NH), lambda i: (0, 0)),
            pl.BlockSpec((DPAD, NH), lambda i: (0, 0)),
            pl.BlockSpec((NH, NH), lambda i: (0, 0)),
            pl.BlockSpec((1, NH), lambda i: (0, 0)),
            pl.BlockSpec((NH, 128), lambda i: (0, 0)),
            pl.BlockSpec((1, 128), lambda i: (0, 0)),
        ],
        out_specs=pl.BlockSpec((bm, 128), lambda i: (i, 0)),
        out_shape=jax.ShapeDtypeStruct((m, 128), jnp.float32),
    )(xs, xd, ea, w1a, w1b, w1c, b1.reshape(1, NH), w2p, b2p)


def _padw(w):
    return jnp.concatenate([w, jnp.zeros((DPAD - NH, w.shape[1]), w.dtype)])


def kernel(x, edge_index, edge_attr, pos_edge_index, pos_edge_attr,
           neg_edge_index, neg_edge_attr, params):
    e = edge_index.shape[1]
    e_lp = pos_edge_index.shape[1]
    src, dst = edge_index[0], edge_index[1]

    x0 = _mm(x, params["node_emb"]["w"], params["node_emb"]["b"])
    eattr = _mm(edge_attr, params["edge_emb"]["w"], params["edge_emb"]["b"])
    pos_ea = _mm(pos_edge_attr, params["edge_emb"]["w"],
                 params["edge_emb"]["b"])
    neg_ea = _mm(neg_edge_attr, params["edge_emb"]["w"],
                 params["edge_emb"]["b"])

    idx_layer = _pad_idx(jnp.concatenate([dst, src]), NW * GW)
    idx_heads = _pad_idx(
        jnp.concatenate([pos_edge_index[0], pos_edge_index[1],
                         neg_edge_index[0], neg_edge_index[1]]), NW * GW)

    eb = 2000
    dst3 = dst.reshape(e // eb, 1, eb)
    half_fill = (jnp.arange(e, dtype=jnp.int32) % DUMP) + HALF
    idx_seg = jnp.concatenate([
        jnp.where(dst < HALF, dst, half_fill),
        jnp.where(dst >= HALF, dst - HALF, half_fill),
    ])

    xt = x0
    xt_pad = jnp.concatenate(
        [xt, jnp.zeros((N_NODES, DPAD - NH), jnp.float32)], axis=-1)
    g = None
    for lp in params["layers"]:
        if g is None:
            g = _sc_gather(xt_pad, idx_layer)
        xd_g, xs_g = g[:e], g[e:2 * e]

        wpre = jnp.concatenate([p["w"] for p in lp["pre"]], axis=1)
        bpre = jnp.concatenate([p["b"] for p in lp["pre"]])
        wpre = jnp.pad(wpre, ((0, 0), (0, FW - T * F_IN)))
        bpre = jnp.pad(bpre, (0, FW - T * F_IN))
        wd, ws, we = wpre[:NH], wpre[NH:2 * NH], wpre[2 * NH:]
        wee = lp["edge_enc"]["w"] @ we
        bee = lp["edge_enc"]["b"] @ we + bpre
        msgs, msgs2 = _mm3_sq(xd_g, xs_g, eattr, _padw(wd), _padw(ws), wee,
                              bee)

        sums8 = _sc_segsum(msgs, msgs2, idx_seg)
        mn_a, mx_a, cnt = _minmax_cnt(msgs, dst3, eb)
        rows = jnp.concatenate(
            [sums8[:, 0], sums8[:, 1, :N_NODES - HALF]], axis=1)
        sum_full = jnp.concatenate([rows[0], rows[1], rows[2], rows[3]],
                                   axis=-1)
        sq_full = jnp.concatenate([rows[4], rows[5], rows[6], rows[7]],
                                  axis=-1)

        pwb = jnp.zeros((T * 1300, T * F_OUT), jnp.float32)
        for t in range(T):
            pwb = pwb.at[t * 1300:(t + 1) * 1300,
                         t * F_OUT:(t + 1) * F_OUT].set(lp["post"][t]["w"])
        pbb = jnp.concatenate([p["b"] for p in lp["post"]])

        h, st = _post(xt, sum_full, sq_full, mn_a, mx_a, cnt, pwb, pbb,
                      lp["lin"]["w"], lp["lin"]["b"])
        xt, xt_pad = _bn_res(xt, h, st, lp["bn_g"], lp["bn_b"])

        g = _sc_gather(xt_pad, idx_layer)
        xd2, xs2 = g[:e], g[e:2 * e]
        w1 = lp["emlp1"]["w"]
        z = _mm3(xs2, xd2, eattr, _padw(w1[:NH]), _padw(w1[NH:2 * NH]),
                 w1[2 * NH:], lp["emlp1"]["b"], relu=True)
        eattr = _mm_res(z, lp["emlp2"]["w"], lp["emlp2"]["b"], eattr)

    gh = _sc_gather(xt_pad, idx_heads)
    ps, pd = gh[:e_lp], gh[e_lp:2 * e_lp]
    ns, nd = gh[2 * e_lp:3 * e_lp], gh[3 * e_lp:4 * e_lp]
    d1 = params["dec1"]["w"]
    pos_out = _head(ps, pd, pos_ea, _padw(d1[:NH]), _padw(d1[NH:2 * NH]),
                    d1[2 * NH:], params["dec1"]["b"],
                    params["dec2"]["w"], params["dec2"]["b"])
    neg_out = _head(ns, nd, neg_ea, _padw(d1[:NH]), _padw(d1[NH:2 * NH]),
                    d1[2 * NH:], params["dec1"]["b"],
                    params["dec2"]["w"], params["dec2"]["b"])
    return pos_out[:, 0], neg_out[:, 0]


# speedup vs baseline: 28.3392x; 1.0005x over previous
import functools
import math

import jax
import jax.numpy as jnp
from jax import lax
from jax.experimental import pallas as pl
from jax.experimental.pallas import tpu as pltpu
from jax.experimental.pallas import tpu_sc as plsc

N_NODES = 10000
T = 5
F_IN = 100
NH = 100
F_OUT = 20
DPAD = 128
NW = 32
GW = 128
AGG_FCHUNK = 256
FW = 512
AVG_LOG = math.log(17.0)


def _sc_gather_body(table_hbm, idx_hbm, out_hbm, idx_v, rows_v, sem):
    bpw = idx_v.shape[0]
    wid = lax.axis_index("s") * 2 + lax.axis_index("c")
    base = wid * bpw
    pltpu.sync_copy(idx_hbm.at[pl.ds(base, bpw)], idx_v)
    for w in range(bpw // GW):
        pltpu.async_copy(
            table_hbm.at[idx_v.at[pl.ds(w * GW, GW)]], rows_v, sem
        ).wait()
        pltpu.sync_copy(rows_v, out_hbm.at[pl.ds(base + w * GW, GW)])


def _sc_gather(table, idx):
    b = idx.shape[0]
    bpw = b // NW
    mesh = plsc.VectorSubcoreMesh(core_axis_name="c", subcore_axis_name="s")
    kern = pl.kernel(
        _sc_gather_body,
        out_type=jax.ShapeDtypeStruct((b, DPAD), jnp.float32),
        mesh=mesh,
        scratch_types=[
            pltpu.VMEM((bpw,), jnp.int32),
            pltpu.VMEM((GW, DPAD), jnp.float32),
            pltpu.SemaphoreType.DMA,
        ],
    )
    return kern(table, idx)


SEG_WIN = 80
SEG_NWIN = 125
HALF = 5120
DUMP = 8


def _sc_segsum_body(m_hbm, m2_hbm, idx_hbm, out_hbm,
                    zero_v, idx_a, idx_b, rows_a, rows_b,
                    acc_sh, sems):
    c = lax.axis_index("c")
    s = lax.axis_index("s")
    e_tot = SEG_WIN * SEG_NWIN * 16
    nrows = HALF // 16
    row0 = pl.multiple_of(s * nrows, 8)

    @pl.loop(0, nrows)
    def _(r):
        for k in range(8):
            zero_v[r, pl.ds(k * 16, 16)] = jnp.zeros((16,), jnp.float32)

    for r in range(8):
        src = m_hbm if r < 4 else m2_hbm
        q_out = r
        col0 = (r % 4) * 128

        pltpu.sync_copy(zero_v, acc_sh.at[pl.ds(row0, nrows)])
        plsc.subcore_barrier()

        bufs = ((idx_a, rows_a, sems.at[0], sems.at[1]),
                (idx_b, rows_b, sems.at[2], sems.at[3]))

        def issue(w, bi):
            idx_v, rows_v, sem_i, sem_r = bufs[bi]
            j = s * SEG_NWIN + w
            roff = pl.multiple_of(j * SEG_WIN, 8)
            ioff = pl.multiple_of(c * e_tot + j * SEG_WIN, 8)
            h1 = pltpu.async_copy(idx_hbm.at[pl.ds(ioff, SEG_WIN)], idx_v,
                                  sem_i)
            h2 = pltpu.async_copy(
                src.at[pl.ds(roff, SEG_WIN), pl.ds(col0, 128)],
                rows_v, sem_r)
            return h1, h2

        def pair(w):
            hs = (issue(w, 0), issue(w + 1, 1))
            for b in range(2):
                idx_v, rows_v, _, _ = bufs[b]
                hs[b][0].wait()
                hs[b][1].wait()
                pltpu.sync_copy(rows_v, acc_sh.at[idx_v], add=True)

        @pl.loop(0, SEG_NWIN - 1, step=2)
        def _(w):
            pair(w)

        hl = issue(SEG_NWIN - 1, 0)
        hl[0].wait()
        hl[1].wait()
        pltpu.sync_copy(rows_a, acc_sh.at[idx_a], add=True)

        plsc.subcore_barrier()
        pltpu.sync_copy(acc_sh.at[pl.ds(row0, nrows)],
                        out_hbm.at[q_out].at[c].at[pl.ds(row0, nrows)])
        plsc.subcore_barrier()


def _sc_segsum(msgs, msgs2, idx_flat):
    mesh = plsc.VectorSubcoreMesh(core_axis_name="c", subcore_axis_name="s")
    kern = pl.kernel(
        _sc_segsum_body,
        out_type=jax.ShapeDtypeStruct((8, 2, HALF, 128), jnp.float32),
        mesh=mesh,
        scratch_types=[
            pltpu.VMEM((HALF // 16, 128), jnp.float32),
            pltpu.VMEM((SEG_WIN,), jnp.int32),
            pltpu.VMEM((SEG_WIN,), jnp.int32),
            pltpu.VMEM((SEG_WIN, 128), jnp.float32),
            pltpu.VMEM((SEG_WIN, 128), jnp.float32),
            pltpu.VMEM_SHARED((HALF + DUMP, 128), jnp.float32),
            pltpu.SemaphoreType.DMA((4,)),
        ],
    )
    return kern(msgs, msgs2, idx_flat)


def _pad_idx(idx, mult):
    b = idx.shape[0]
    pad = (-b) % mult
    if pad:
        fill = jnp.arange(pad, dtype=jnp.int32) % N_NODES
        idx = jnp.concatenate([idx, fill])
    return idx


def _mm_body(x_ref, w_ref, b_ref, o_ref, *, relu):
    acc = jnp.dot(x_ref[...], w_ref[...], preferred_element_type=jnp.float32)
    acc = acc + b_ref[...]
    if relu:
        acc = jnp.maximum(acc, 0.0)
    o_ref[...] = acc


def _mm(x, w, b, relu=False, bm=1000):
    m, k = x.shape
    n = w.shape[1]
    body = functools.partial(_mm_body, relu=relu)
    return pl.pallas_call(
        body,
        grid=(m // bm,),
        in_specs=[
            pl.BlockSpec((bm, k), lambda i: (i, 0)),
            pl.BlockSpec((k, n), lambda i: (0, 0)),
            pl.BlockSpec((1, n), lambda i: (0, 0)),
        ],
        out_specs=pl.BlockSpec((bm, n), lambda i: (i, 0)),
        out_shape=jax.ShapeDtypeStruct((m, n), jnp.float32),
    )(x, w, b.reshape(1, n))


def _mm_res_body(x_ref, w_ref, b_ref, res_ref, o_ref):
    acc = jnp.dot(x_ref[...], w_ref[...], preferred_element_type=jnp.float32)
    o_ref[...] = res_ref[...] + (acc + b_ref[...]) * 0.5


def _mm_res(x, w, b, res, bm=1000):
    m, k = x.shape
    n = w.shape[1]
    return pl.pallas_call(
        _mm_res_body,
        grid=(m // bm,),
        in_specs=[
            pl.BlockSpec((bm, k), lambda i: (i, 0)),
            pl.BlockSpec((k, n), lambda i: (0, 0)),
            pl.BlockSpec((1, n), lambda i: (0, 0)),
            pl.BlockSpec((bm, n), lambda i: (i, 0)),
        ],
        out_specs=pl.BlockSpec((bm, n), lambda i: (i, 0)),
        out_shape=jax.ShapeDtypeStruct((m, n), jnp.float32),
    )(x, w, b.reshape(1, n), res)


def _mm3_body(a_ref, b_ref, c_ref, wa_ref, wb_ref, wc_ref, bias_ref,
              o_ref, *, relu):
    acc = jnp.dot(a_ref[...], wa_ref[...], preferred_element_type=jnp.float32)
    acc += jnp.dot(b_ref[...], wb_ref[...], preferred_element_type=jnp.float32)
    acc += jnp.dot(c_ref[...], wc_ref[...], preferred_element_type=jnp.float32)
    acc = acc + bias_ref[...]
    if relu:
        acc = jnp.maximum(acc, 0.0)
    o_ref[...] = acc


def _mm3_sq_body(a_ref, b_ref, c_ref, wa_ref, wb_ref, wc_ref, bias_ref,
                 o_ref, o2_ref):
    acc = jnp.dot(a_ref[...], wa_ref[...], preferred_element_type=jnp.float32)
    acc += jnp.dot(b_ref[...], wb_ref[...], preferred_element_type=jnp.float32)
    acc += jnp.dot(c_ref[...], wc_ref[...], preferred_element_type=jnp.float32)
    acc = acc + bias_ref[...]
    o_ref[...] = acc
    o2_ref[...] = acc * acc


def _mm3_sq(a, b, c, wa, wb, wc, bias, bm=1000):
    m = a.shape[0]
    n = wa.shape[1]
    return pl.pallas_call(
        _mm3_sq_body,
        grid=(m // bm,),
        in_specs=[
            pl.BlockSpec((bm, a.shape[1]), lambda i: (i, 0)),
            pl.BlockSpec((bm, b.shape[1]), lambda i: (i, 0)),
            pl.BlockSpec((bm, c.shape[1]), lambda i: (i, 0)),
            pl.BlockSpec((a.shape[1], n), lambda i: (0, 0)),
            pl.BlockSpec((b.shape[1], n), lambda i: (0, 0)),
            pl.BlockSpec((c.shape[1], n), lambda i: (0, 0)),
            pl.BlockSpec((1, n), lambda i: (0, 0)),
        ],
        out_specs=[pl.BlockSpec((bm, n), lambda i: (i, 0))] * 2,
        out_shape=[jax.ShapeDtypeStruct((m, n), jnp.float32)] * 2,
    )(a, b, c, wa, wb, wc, bias.reshape(1, n))


def _mm3(a, b, c, wa, wb, wc, bias, relu=False, bm=1000):
    m = a.shape[0]
    n = wa.shape[1]
    body = functools.partial(_mm3_body, relu=relu)
    return pl.pallas_call(
        body,
        grid=(m // bm,),
        in_specs=[
            pl.BlockSpec((bm, a.shape[1]), lambda i: (i, 0)),
            pl.BlockSpec((bm, b.shape[1]), lambda i: (i, 0)),
            pl.BlockSpec((bm, c.shape[1]), lambda i: (i, 0)),
            pl.BlockSpec((a.shape[1], n), lambda i: (0, 0)),
            pl.BlockSpec((b.shape[1], n), lambda i: (0, 0)),
            pl.BlockSpec((c.shape[1], n), lambda i: (0, 0)),
            pl.BlockSpec((1, n), lambda i: (0, 0)),
        ],
        out_specs=pl.BlockSpec((bm, n), lambda i: (i, 0)),
        out_shape=jax.ShapeDtypeStruct((m, n), jnp.float32),
    )(a, b, c, wa, wb, wc, bias.reshape(1, n))


def _minmax_body(dst_ref, msgs_ref, mn_ref, mx_ref, cnt_ref, *, eb,
                 with_cnt):
    c = pl.program_id(0)

    @pl.when(c == 0)
    def _init():
        mn_ref[...] = jnp.full_like(mn_ref, jnp.inf)
        mx_ref[...] = jnp.full_like(mx_ref, -jnp.inf)
        if with_cnt:
            cnt_ref[...] = jnp.zeros_like(cnt_ref)

    def step(e, _):
        d = dst_ref[0, 0, e]
        row = msgs_ref[pl.ds(e, 1), :]
        mn_ref[pl.ds(d, 1), :] = jnp.minimum(mn_ref[pl.ds(d, 1), :], row)
        mx_ref[pl.ds(d, 1), :] = jnp.maximum(mx_ref[pl.ds(d, 1), :], row)
        if with_cnt:
            cnt_ref[pl.ds(d, 1), :] += 1.0
        return 0

    lax.fori_loop(0, eb, step, 0)


def _minmax_cnt(msgs, dst3, eb, with_cnt):
    e_tot = msgs.shape[0]
    fw = msgs.shape[1]
    body = functools.partial(_minmax_body, eb=eb, with_cnt=with_cnt)
    cnt_shape = (N_NODES, 8) if with_cnt else (8, 8)
    return pl.pallas_call(
        body,
        grid=(e_tot // eb,),
        in_specs=[
            pl.BlockSpec((1, 1, eb), lambda c: (c, 0, 0),
                         memory_space=pltpu.SMEM),
            pl.BlockSpec((eb, fw), lambda c: (c, 0)),
        ],
        out_specs=[
            pl.BlockSpec((N_NODES, fw), lambda c: (0, 0)),
            pl.BlockSpec((N_NODES, fw), lambda c: (0, 0)),
            pl.BlockSpec(cnt_shape, lambda c: (0, 0)),
        ],
        out_shape=[
            jax.ShapeDtypeStruct((N_NODES, fw), jnp.float32),
            jax.ShapeDtypeStruct((N_NODES, fw), jnp.float32),
            jax.ShapeDtypeStruct(cnt_shape, jnp.float32),
        ],
    )(dst3, msgs)


def _agg_body(dst_ref, msgs_ref, sum_ref, sq_ref, mn_ref, mx_ref, cnt_ref,
              *, eb, with_cnt):
    c = pl.program_id(0)

    @pl.when(c == 0)
    def _init():
        sum_ref[...] = jnp.zeros_like(sum_ref)
        sq_ref[...] = jnp.zeros_like(sq_ref)
        mn_ref[...] = jnp.full_like(mn_ref, jnp.inf)
        mx_ref[...] = jnp.full_like(mx_ref, -jnp.inf)
        if with_cnt:
            cnt_ref[...] = jnp.zeros_like(cnt_ref)

    def step(e, _):
        d = dst_ref[0, 0, e]
        row = msgs_ref[pl.ds(e, 1), :]
        sum_ref[pl.ds(d, 1), :] += row
        sq_ref[pl.ds(d, 1), :] += row * row
        mn_ref[pl.ds(d, 1), :] = jnp.minimum(mn_ref[pl.ds(d, 1), :], row)
        mx_ref[pl.ds(d, 1), :] = jnp.maximum(mx_ref[pl.ds(d, 1), :], row)
        if with_cnt:
            cnt_ref[pl.ds(d, 1), :] += 1.0
        return 0

    lax.fori_loop(0, eb, step, 0)


def _aggregate(msgs, dst3, eb):
    e_tot = msgs.shape[0]
    fw = msgs.shape[1]
    nfc = fw // AGG_FCHUNK
    outs = []
    cnt = None
    for p in range(nfc):
        with_cnt = p == 0
        body = functools.partial(_agg_body, eb=eb, with_cnt=with_cnt)
        agg_spec = pl.BlockSpec((N_NODES, AGG_FCHUNK), lambda c: (0, 0))
        cnt_shape = (N_NODES, 8) if with_cnt else (8, 8)
        res = pl.pallas_call(
            body,
            grid=(e_tot // eb,),
            in_specs=[
                pl.BlockSpec((1, 1, eb), lambda c: (c, 0, 0),
                             memory_space=pltpu.SMEM),
                pl.BlockSpec((eb, AGG_FCHUNK), lambda c, _p=p: (c, _p)),
            ],
            out_specs=[agg_spec] * 4 + [
                pl.BlockSpec(cnt_shape, lambda c: (0, 0))],
            out_shape=[jax.ShapeDtypeStruct((N_NODES, AGG_FCHUNK),
                                            jnp.float32)] * 4 + [
                jax.ShapeDtypeStruct(cnt_shape, jnp.float32)],
        )(dst3, msgs)
        outs.append(res[:4])
        if with_cnt:
            cnt = res[4]
    return outs, cnt


def _post_body(x_ref, sum_ref, sq_ref, mn_ref, mx_ref, cnt_ref,
               pw_ref, pb_ref, lw_ref, lb_ref, h_ref, st_ref):
    i = pl.program_id(0)
    sum_full = sum_ref[...]
    sq_full = sq_ref[...]
    mn_full = mn_ref[...]
    mx_full = mx_ref[...]
    cnt = cnt_ref[:, 0:1]
    cnt_c = jnp.maximum(cnt, 1.0)
    inv = 1.0 / cnt_c
    has = cnt > 0.0
    lg = jnp.log(cnt_c + 1.0)
    amp = lg * (1.0 / AVG_LOG)
    att = AVG_LOG / lg

    feats = []
    for t in range(T):
        s = t * F_IN
        mean = sum_full[:, s:s + F_IN] * inv
        mean2 = sq_full[:, s:s + F_IN] * inv
        std = jnp.sqrt(jnp.maximum(mean2 - mean * mean, 0.0) + 1e-5)
        mnv = jnp.where(has, mn_full[:, s:s + F_IN], 0.0)
        mxv = jnp.where(has, mx_full[:, s:s + F_IN], 0.0)
        agg = jnp.concatenate([mean, mnv, mxv, std], axis=-1)
        feats.append(jnp.concatenate(
            [x_ref[...], agg, agg * amp, agg * att], axis=-1))
    hcat = jnp.concatenate(feats, axis=-1)
    out = jnp.dot(hcat, pw_ref[...], preferred_element_type=jnp.float32)
    out = out + pb_ref[...]
    h = jnp.dot(out, lw_ref[...], preferred_element_type=jnp.float32)
    h = h + lb_ref[...]
    h_ref[...] = h

    @pl.when(i == 0)
    def _():
        st_ref[...] = jnp.zeros_like(st_ref)

    st_ref[0:1, 0:NH] += jnp.sum(h, axis=0, keepdims=True)
    st_ref[1:2, 0:NH] += jnp.sum(h * h, axis=0, keepdims=True)


def _post(x, sum_full, sq_full, mn_a, mx_a, cnt, pw, pb, lw, lb, bm=400):
    return pl.pallas_call(
        _post_body,
        grid=(N_NODES // bm,),
        in_specs=[
            pl.BlockSpec((bm, NH), lambda i: (i, 0)),
        ] + [
            pl.BlockSpec((bm, FW), lambda i: (i, 0))
            for _ in range(4)
        ] + [
            pl.BlockSpec((bm, 8), lambda i: (i, 0)),
            pl.BlockSpec((T * 1300, T * F_OUT), lambda i: (0, 0)),
            pl.BlockSpec((1, T * F_OUT), lambda i: (0, 0)),
            pl.BlockSpec((NH, NH), lambda i: (0, 0)),
            pl.BlockSpec((1, NH), lambda i: (0, 0)),
        ],
        out_specs=[
            pl.BlockSpec((bm, NH), lambda i: (i, 0)),
            pl.BlockSpec((8, 128), lambda i: (0, 0)),
        ],
        out_shape=[
            jax.ShapeDtypeStruct((N_NODES, NH), jnp.float32),
            jax.ShapeDtypeStruct((8, 128), jnp.float32),
        ],
    )(x, sum_full, sq_full, mn_a, mx_a, cnt, pw, pb.reshape(1, T * F_OUT), lw,
      lb.reshape(1, NH))


def _bn_res_body(x_ref, h_ref, st_ref, g_ref, b_ref, o_ref, op_ref):
    mean = st_ref[0:1, 0:NH] * (1.0 / N_NODES)
    var = st_ref[1:2, 0:NH] * (1.0 / N_NODES) - mean * mean
    rstd = lax.rsqrt(var + 1e-5)
    hn = g_ref[...] * (h_ref[...] - mean) * rstd + b_ref[...]
    xn = (x_ref[...] + jnp.maximum(hn, 0.0)) * 0.5
    o_ref[...] = xn
    op_ref[...] = jnp.concatenate(
        [xn, jnp.zeros((xn.shape[0], DPAD - NH), jnp.float32)], axis=-1)


def _bn_res(x, h, st, g, b, bm=1000):
    return pl.pallas_call(
        _bn_res_body,
        grid=(N_NODES // bm,),
        in_specs=[
            pl.BlockSpec((bm, NH), lambda i: (i, 0)),
            pl.BlockSpec((bm, NH), lambda i: (i, 0)),
            pl.BlockSpec((8, 128), lambda i: (0, 0)),
            pl.BlockSpec((1, NH), lambda i: (0, 0)),
            pl.BlockSpec((1, NH), lambda i: (0, 0)),
        ],
        out_specs=[
            pl.BlockSpec((bm, NH), lambda i: (i, 0)),
            pl.BlockSpec((bm, DPAD), lambda i: (i, 0)),
        ],
        out_shape=[
            jax.ShapeDtypeStruct((N_NODES, NH), jnp.float32),
            jax.ShapeDtypeStruct((N_NODES, DPAD), jnp.float32),
        ],
    )(x, h, st, g.reshape(1, NH), b.reshape(1, NH))


def _head_body(a_ref, b_ref, c_ref, w1a_ref, w1b_ref, w1c_ref, b1_ref,
               w2_ref, b2_ref, o_ref):
    z = jnp.dot(a_ref[...], w1a_ref[...], preferred_element_type=jnp.float32)
    z += jnp.dot(b_ref[...], w1b_ref[...], preferred_element_type=jnp.float32)
    z += jnp.dot(c_ref[...], w1c_ref[...], preferred_element_type=jnp.float32)
    z = jnp.maximum(z + b1_ref[...], 0.0)
    y = jnp.dot(z, w2_ref[...], preferred_element_type=jnp.float32)
    o_ref[...] = jax.nn.sigmoid(y + b2_ref[...])


def _head(xs, xd, ea, w1a, w1b, w1c, b1, w2, b2, bm=1000):
    m = xs.shape[0]
    w2p = jnp.concatenate([w2, jnp.zeros((NH, 127), jnp.float32)], axis=1)
    b2p = jnp.concatenate([b2, jnp.zeros((127,), jnp.float32)]).reshape(1, 128)
    return pl.pallas_call(
        _head_body,
        grid=(m // bm,),
        in_specs=[
            pl.BlockSpec((bm, DPAD), lambda i: (i, 0)),
            pl.BlockSpec((bm, DPAD), lambda i: (i, 0)),
            pl.BlockSpec((bm, NH), lambda i: (i, 0)),
            pl.BlockSpec((DPAD, NH), lambda i: (0, 0)),
            pl.BlockSpec((DPAD, NH), lambda i: (0, 0)),
            pl.BlockSpec((NH, NH), lambda i: (0, 0)),
            pl.BlockSpec((1, NH), lambda i: (0, 0)),
            pl.BlockSpec((NH, 128), lambda i: (0, 0)),
            pl.BlockSpec((1, 128), lambda i: (0, 0)),
        ],
        out_specs=pl.BlockSpec((bm, 128), lambda i: (i, 0)),
        out_shape=jax.ShapeDtypeStruct((m, 128), jnp.float32),
    )(xs, xd, ea, w1a, w1b, w1c, b1.reshape(1, NH), w2p, b2p)


def _padw(w):
    return jnp.concatenate([w, jnp.zeros((DPAD - NH, w.shape[1]), w.dtype)])


def kernel(x, edge_index, edge_attr, pos_edge_index, pos_edge_attr,
           neg_edge_index, neg_edge_attr, params):
    e = edge_index.shape[1]
    e_lp = pos_edge_index.shape[1]
    src, dst = edge_index[0], edge_index[1]

    x0 = _mm(x, params["node_emb"]["w"], params["node_emb"]["b"])
    eattr = _mm(edge_attr, params["edge_emb"]["w"], params["edge_emb"]["b"])
    pos_ea = _mm(pos_edge_attr, params["edge_emb"]["w"],
                 params["edge_emb"]["b"])
    neg_ea = _mm(neg_edge_attr, params["edge_emb"]["w"],
                 params["edge_emb"]["b"])

    idx_layer = _pad_idx(jnp.concatenate([dst, src]), NW * GW)
    idx_heads = _pad_idx(
        jnp.concatenate([pos_edge_index[0], pos_edge_index[1],
                         neg_edge_index[0], neg_edge_index[1]]), NW * GW)

    eb = 2000
    dst3 = dst.reshape(e // eb, 1, eb)
    half_fill = (jnp.arange(e, dtype=jnp.int32) % DUMP) + HALF
    idx_seg = jnp.concatenate([
        jnp.where(dst < HALF, dst, half_fill),
        jnp.where(dst >= HALF, dst - HALF, half_fill),
    ])

    xt = x0
    xt_pad = jnp.concatenate(
        [xt, jnp.zeros((N_NODES, DPAD - NH), jnp.float32)], axis=-1)
    g = None
    cnt = None
    for lp in params["layers"]:
        if g is None:
            g = _sc_gather(xt_pad, idx_layer)
        xd_g, xs_g = g[:e], g[e:2 * e]

        wpre = jnp.concatenate([p["w"] for p in lp["pre"]], axis=1)
        bpre = jnp.concatenate([p["b"] for p in lp["pre"]])
        wpre = jnp.pad(wpre, ((0, 0), (0, FW - T * F_IN)))
        bpre = jnp.pad(bpre, (0, FW - T * F_IN))
        wd, ws, we = wpre[:NH], wpre[NH:2 * NH], wpre[2 * NH:]
        wee = lp["edge_enc"]["w"] @ we
        bee = lp["edge_enc"]["b"] @ we + bpre
        msgs, msgs2 = _mm3_sq(xd_g, xs_g, eattr, _padw(wd), _padw(ws), wee,
                              bee)

        sums8 = _sc_segsum(msgs, msgs2, idx_seg)
        mn_a, mx_a, cnt_new = _minmax_cnt(msgs, dst3, eb, with_cnt=cnt is None)
        if cnt is None:
            cnt = cnt_new
        rows = jnp.concatenate(
            [sums8[:, 0], sums8[:, 1, :N_NODES - HALF]], axis=1)
        sum_full = jnp.concatenate([rows[0], rows[1], rows[2], rows[3]],
                                   axis=-1)
        sq_full = jnp.concatenate([rows[4], rows[5], rows[6], rows[7]],
                                  axis=-1)

        pwb = jnp.zeros((T * 1300, T * F_OUT), jnp.float32)
        for t in range(T):
            pwb = pwb.at[t * 1300:(t + 1) * 1300,
                         t * F_OUT:(t + 1) * F_OUT].set(lp["post"][t]["w"])
        pbb = jnp.concatenate([p["b"] for p in lp["post"]])

        h, st = _post(xt, sum_full, sq_full, mn_a, mx_a, cnt, pwb, pbb,
                      lp["lin"]["w"], lp["lin"]["b"])
        xt, xt_pad = _bn_res(xt, h, st, lp["bn_g"], lp["bn_b"])

        g = _sc_gather(xt_pad, idx_layer)
        xd2, xs2 = g[:e], g[e:2 * e]
        w1 = lp["emlp1"]["w"]
        z = _mm3(xs2, xd2, eattr, _padw(w1[:NH]), _padw(w1[NH:2 * NH]),
                 w1[2 * NH:], lp["emlp1"]["b"], relu=True)
        eattr = _mm_res(z, lp["emlp2"]["w"], lp["emlp2"]["b"], eattr)

    gh = _sc_gather(xt_pad, idx_heads)
    ps, pd = gh[:e_lp], gh[e_lp:2 * e_lp]
    ns, nd = gh[2 * e_lp:3 * e_lp], gh[3 * e_lp:4 * e_lp]
    d1 = params["dec1"]["w"]
    pos_out = _head(ps, pd, pos_ea, _padw(d1[:NH]), _padw(d1[NH:2 * NH]),
                    d1[2 * NH:], params["dec1"]["b"],
                    params["dec2"]["w"], params["dec2"]["b"])
    neg_out = _head(ns, nd, neg_ea, _padw(d1[:NH]), _padw(d1[NH:2 * NH]),
                    d1[2 * NH:], params["dec1"]["b"],
                    params["dec2"]["w"], params["dec2"]["b"])
    return pos_out[:, 0], neg_out[:, 0]


# speedup vs baseline: 28.3437x; 1.0002x over previous
import functools
import math

import jax
import jax.numpy as jnp
from jax import lax
from jax.experimental import pallas as pl
from jax.experimental.pallas import tpu as pltpu
from jax.experimental.pallas import tpu_sc as plsc

N_NODES = 10000
T = 5
F_IN = 100
NH = 100
F_OUT = 20
DPAD = 128
NW = 32
GW = 128
FW = 512
AVG_LOG = math.log(17.0)


def _sc_gather_body(table_hbm, idx_hbm, out_hbm, idx_v, rows_v, sem):
    bpw = idx_v.shape[0]
    wid = lax.axis_index("s") * 2 + lax.axis_index("c")
    base = wid * bpw
    pltpu.sync_copy(idx_hbm.at[pl.ds(base, bpw)], idx_v)
    for w in range(bpw // GW):
        pltpu.async_copy(
            table_hbm.at[idx_v.at[pl.ds(w * GW, GW)]], rows_v, sem
        ).wait()
        pltpu.sync_copy(rows_v, out_hbm.at[pl.ds(base + w * GW, GW)])


def _sc_gather(table, idx):
    b = idx.shape[0]
    bpw = b // NW
    mesh = plsc.VectorSubcoreMesh(core_axis_name="c", subcore_axis_name="s")
    kern = pl.kernel(
        _sc_gather_body,
        out_type=jax.ShapeDtypeStruct((b, DPAD), jnp.float32),
        mesh=mesh,
        scratch_types=[
            pltpu.VMEM((bpw,), jnp.int32),
            pltpu.VMEM((GW, DPAD), jnp.float32),
            pltpu.SemaphoreType.DMA,
        ],
    )
    return kern(table, idx)


SEG_WIN = 80
SEG_NWIN = 125
HALF = 5120
DUMP = 8


def _sc_segsum_body(m_hbm, m2_hbm, idx_hbm, out_hbm,
                    zero_v, idx_a, idx_b, rows_a, rows_b,
                    acc_sh, sems):
    c = lax.axis_index("c")
    s = lax.axis_index("s")
    e_tot = SEG_WIN * SEG_NWIN * 16
    nrows = HALF // 16
    row0 = pl.multiple_of(s * nrows, 8)

    @pl.loop(0, nrows)
    def _(r):
        for k in range(8):
            zero_v[r, pl.ds(k * 16, 16)] = jnp.zeros((16,), jnp.float32)

    for r in range(8):
        src = m_hbm if r < 4 else m2_hbm
        q_out = r
        col0 = (r % 4) * 128

        pltpu.sync_copy(zero_v, acc_sh.at[pl.ds(row0, nrows)])
        plsc.subcore_barrier()

        bufs = ((idx_a, rows_a, sems.at[0], sems.at[1]),
                (idx_b, rows_b, sems.at[2], sems.at[3]))

        def issue(w, bi):
            idx_v, rows_v, sem_i, sem_r = bufs[bi]
            j = s * SEG_NWIN + w
            roff = pl.multiple_of(j * SEG_WIN, 8)
            ioff = pl.multiple_of(c * e_tot + j * SEG_WIN, 8)
            h1 = pltpu.async_copy(idx_hbm.at[pl.ds(ioff, SEG_WIN)], idx_v,
                                  sem_i)
            h2 = pltpu.async_copy(
                src.at[pl.ds(roff, SEG_WIN), pl.ds(col0, 128)],
                rows_v, sem_r)
            return h1, h2

        def pair(w):
            hs = (issue(w, 0), issue(w + 1, 1))
            for b in range(2):
                idx_v, rows_v, _, _ = bufs[b]
                hs[b][0].wait()
                hs[b][1].wait()
                pltpu.sync_copy(rows_v, acc_sh.at[idx_v], add=True)

        @pl.loop(0, SEG_NWIN - 1, step=2)
        def _(w):
            pair(w)

        hl = issue(SEG_NWIN - 1, 0)
        hl[0].wait()
        hl[1].wait()
        pltpu.sync_copy(rows_a, acc_sh.at[idx_a], add=True)

        plsc.subcore_barrier()
        pltpu.sync_copy(acc_sh.at[pl.ds(row0, nrows)],
                        out_hbm.at[q_out].at[c].at[pl.ds(row0, nrows)])
        plsc.subcore_barrier()


def _sc_segsum(msgs, msgs2, idx_flat):
    mesh = plsc.VectorSubcoreMesh(core_axis_name="c", subcore_axis_name="s")
    kern = pl.kernel(
        _sc_segsum_body,
        out_type=jax.ShapeDtypeStruct((8, 2, HALF, 128), jnp.float32),
        mesh=mesh,
        scratch_types=[
            pltpu.VMEM((HALF // 16, 128), jnp.float32),
            pltpu.VMEM((SEG_WIN,), jnp.int32),
            pltpu.VMEM((SEG_WIN,), jnp.int32),
            pltpu.VMEM((SEG_WIN, 128), jnp.float32),
            pltpu.VMEM((SEG_WIN, 128), jnp.float32),
            pltpu.VMEM_SHARED((HALF + DUMP, 128), jnp.float32),
            pltpu.SemaphoreType.DMA((4,)),
        ],
    )
    return kern(msgs, msgs2, idx_flat)


def _pad_idx(idx, mult):
    b = idx.shape[0]
    pad = (-b) % mult
    if pad:
        fill = jnp.arange(pad, dtype=jnp.int32) % N_NODES
        idx = jnp.concatenate([idx, fill])
    return idx


def _mm_body(x_ref, w_ref, b_ref, o_ref, *, relu):
    acc = jnp.dot(x_ref[...], w_ref[...], preferred_element_type=jnp.float32)
    acc = acc + b_ref[...]
    if relu:
        acc = jnp.maximum(acc, 0.0)
    o_ref[...] = acc


def _mm(x, w, b, relu=False, bm=1000):
    m, k = x.shape
    n = w.shape[1]
    body = functools.partial(_mm_body, relu=relu)
    return pl.pallas_call(
        body,
        grid=(m // bm,),
        in_specs=[
            pl.BlockSpec((bm, k), lambda i: (i, 0)),
            pl.BlockSpec((k, n), lambda i: (0, 0)),
            pl.BlockSpec((1, n), lambda i: (0, 0)),
        ],
        out_specs=pl.BlockSpec((bm, n), lambda i: (i, 0)),
        out_shape=jax.ShapeDtypeStruct((m, n), jnp.float32),
    )(x, w, b.reshape(1, n))


def _mm_res_body(x_ref, w_ref, b_ref, res_ref, o_ref):
    acc = jnp.dot(x_ref[...], w_ref[...], preferred_element_type=jnp.float32)
    o_ref[...] = res_ref[...] + (acc + b_ref[...]) * 0.5


def _mm_res(x, w, b, res, bm=1000):
    m, k = x.shape
    n = w.shape[1]
    return pl.pallas_call(
        _mm_res_body,
        grid=(m // bm,),
        in_specs=[
            pl.BlockSpec((bm, k), lambda i: (i, 0)),
            pl.BlockSpec((k, n), lambda i: (0, 0)),
            pl.BlockSpec((1, n), lambda i: (0, 0)),
            pl.BlockSpec((bm, n), lambda i: (i, 0)),
        ],
        out_specs=pl.BlockSpec((bm, n), lambda i: (i, 0)),
        out_shape=jax.ShapeDtypeStruct((m, n), jnp.float32),
    )(x, w, b.reshape(1, n), res)


def _mm3_body(a_ref, b_ref, c_ref, wa_ref, wb_ref, wc_ref, bias_ref,
              o_ref, *, relu):
    acc = jnp.dot(a_ref[...], wa_ref[...], preferred_element_type=jnp.float32)
    acc += jnp.dot(b_ref[...], wb_ref[...], preferred_element_type=jnp.float32)
    acc += jnp.dot(c_ref[...], wc_ref[...], preferred_element_type=jnp.float32)
    acc = acc + bias_ref[...]
    if relu:
        acc = jnp.maximum(acc, 0.0)
    o_ref[...] = acc


def _mm3_sq_body(a_ref, b_ref, c_ref, wa_ref, wb_ref, wc_ref, bias_ref,
                 o_ref, o2_ref):
    acc = jnp.dot(a_ref[...], wa_ref[...], preferred_element_type=jnp.float32)
    acc += jnp.dot(b_ref[...], wb_ref[...], preferred_element_type=jnp.float32)
    acc += jnp.dot(c_ref[...], wc_ref[...], preferred_element_type=jnp.float32)
    acc = acc + bias_ref[...]
    o_ref[...] = acc
    o2_ref[...] = acc * acc


def _mm3_sq(a, b, c, wa, wb, wc, bias, bm=1000):
    m = a.shape[0]
    n = wa.shape[1]
    return pl.pallas_call(
        _mm3_sq_body,
        grid=(m // bm,),
        in_specs=[
            pl.BlockSpec((bm, a.shape[1]), lambda i: (i, 0)),
            pl.BlockSpec((bm, b.shape[1]), lambda i: (i, 0)),
            pl.BlockSpec((bm, c.shape[1]), lambda i: (i, 0)),
            pl.BlockSpec((a.shape[1], n), lambda i: (0, 0)),
            pl.BlockSpec((b.shape[1], n), lambda i: (0, 0)),
            pl.BlockSpec((c.shape[1], n), lambda i: (0, 0)),
            pl.BlockSpec((1, n), lambda i: (0, 0)),
        ],
        out_specs=[pl.BlockSpec((bm, n), lambda i: (i, 0))] * 2,
        out_shape=[jax.ShapeDtypeStruct((m, n), jnp.float32)] * 2,
    )(a, b, c, wa, wb, wc, bias.reshape(1, n))


def _mm3(a, b, c, wa, wb, wc, bias, relu=False, bm=1000):
    m = a.shape[0]
    n = wa.shape[1]
    body = functools.partial(_mm3_body, relu=relu)
    return pl.pallas_call(
        body,
        grid=(m // bm,),
        in_specs=[
            pl.BlockSpec((bm, a.shape[1]), lambda i: (i, 0)),
            pl.BlockSpec((bm, b.shape[1]), lambda i: (i, 0)),
            pl.BlockSpec((bm, c.shape[1]), lambda i: (i, 0)),
            pl.BlockSpec((a.shape[1], n), lambda i: (0, 0)),
            pl.BlockSpec((b.shape[1], n), lambda i: (0, 0)),
            pl.BlockSpec((c.shape[1], n), lambda i: (0, 0)),
            pl.BlockSpec((1, n), lambda i: (0, 0)),
        ],
        out_specs=pl.BlockSpec((bm, n), lambda i: (i, 0)),
        out_shape=jax.ShapeDtypeStruct((m, n), jnp.float32),
    )(a, b, c, wa, wb, wc, bias.reshape(1, n))


def _minmax_body(dst_ref, msgs_ref, mn_ref, mx_ref, cnt_ref, *, eb,
                 with_cnt):
    c = pl.program_id(0)

    @pl.when(c == 0)
    def _init():
        mn_ref[...] = jnp.full_like(mn_ref, jnp.inf)
        mx_ref[...] = jnp.full_like(mx_ref, -jnp.inf)
        if with_cnt:
            cnt_ref[...] = jnp.zeros_like(cnt_ref)

    def step(e, _):
        d = dst_ref[0, 0, e]
        row = msgs_ref[pl.ds(e, 1), :]
        mn_ref[pl.ds(d, 1), :] = jnp.minimum(mn_ref[pl.ds(d, 1), :], row)
        mx_ref[pl.ds(d, 1), :] = jnp.maximum(mx_ref[pl.ds(d, 1), :], row)
        if with_cnt:
            cnt_ref[pl.ds(d, 1), :] += 1.0
        return 0

    lax.fori_loop(0, eb, step, 0)


def _minmax_cnt(msgs, dst3, eb, with_cnt):
    e_tot = msgs.shape[0]
    fw = msgs.shape[1]
    body = functools.partial(_minmax_body, eb=eb, with_cnt=with_cnt)
    cnt_shape = (N_NODES, 8) if with_cnt else (8, 8)
    return pl.pallas_call(
        body,
        grid=(e_tot // eb,),
        in_specs=[
            pl.BlockSpec((1, 1, eb), lambda c: (c, 0, 0),
                         memory_space=pltpu.SMEM),
            pl.BlockSpec((eb, fw), lambda c: (c, 0)),
        ],
        out_specs=[
            pl.BlockSpec((N_NODES, fw), lambda c: (0, 0)),
            pl.BlockSpec((N_NODES, fw), lambda c: (0, 0)),
            pl.BlockSpec(cnt_shape, lambda c: (0, 0)),
        ],
        out_shape=[
            jax.ShapeDtypeStruct((N_NODES, fw), jnp.float32),
            jax.ShapeDtypeStruct((N_NODES, fw), jnp.float32),
            jax.ShapeDtypeStruct(cnt_shape, jnp.float32),
        ],
    )(dst3, msgs)


def _post_body(x_ref, sum_ref, sq_ref, mn_ref, mx_ref, cnt_ref,
               pw_ref, pb_ref, lw_ref, lb_ref, h_ref, st_ref):
    i = pl.program_id(0)
    sum_full = sum_ref[...]
    sq_full = sq_ref[...]
    mn_full = mn_ref[...]
    mx_full = mx_ref[...]
    cnt = cnt_ref[:, 0:1]
    cnt_c = jnp.maximum(cnt, 1.0)
    inv = 1.0 / cnt_c
    has = cnt > 0.0
    lg = jnp.log(cnt_c + 1.0)
    amp = lg * (1.0 / AVG_LOG)
    att = AVG_LOG / lg

    feats = []
    for t in range(T):
        s = t * F_IN
        mean = sum_full[:, s:s + F_IN] * inv
        mean2 = sq_full[:, s:s + F_IN] * inv
        std = jnp.sqrt(jnp.maximum(mean2 - mean * mean, 0.0) + 1e-5)
        mnv = jnp.where(has, mn_full[:, s:s + F_IN], 0.0)
        mxv = jnp.where(has, mx_full[:, s:s + F_IN], 0.0)
        agg = jnp.concatenate([mean, mnv, mxv, std], axis=-1)
        feats.append(jnp.concatenate(
            [x_ref[...], agg, agg * amp, agg * att], axis=-1))
    hcat = jnp.concatenate(feats, axis=-1)
    out = jnp.dot(hcat, pw_ref[...], preferred_element_type=jnp.float32)
    out = out + pb_ref[...]
    h = jnp.dot(out, lw_ref[...], preferred_element_type=jnp.float32)
    h = h + lb_ref[...]
    h_ref[...] = h

    @pl.when(i == 0)
    def _():
        st_ref[...] = jnp.zeros_like(st_ref)

    st_ref[0:1, 0:NH] += jnp.sum(h, axis=0, keepdims=True)
    st_ref[1:2, 0:NH] += jnp.sum(h * h, axis=0, keepdims=True)


def _post(x, sum_full, sq_full, mn_a, mx_a, cnt, pw, pb, lw, lb, bm=400):
    return pl.pallas_call(
        _post_body,
        grid=(N_NODES // bm,),
        in_specs=[
            pl.BlockSpec((bm, NH), lambda i: (i, 0)),
        ] + [
            pl.BlockSpec((bm, FW), lambda i: (i, 0))
            for _ in range(4)
        ] + [
            pl.BlockSpec((bm, 8), lambda i: (i, 0)),
            pl.BlockSpec((T * 1300, T * F_OUT), lambda i: (0, 0)),
            pl.BlockSpec((1, T * F_OUT), lambda i: (0, 0)),
            pl.BlockSpec((NH, NH), lambda i: (0, 0)),
            pl.BlockSpec((1, NH), lambda i: (0, 0)),
        ],
        out_specs=[
            pl.BlockSpec((bm, NH), lambda i: (i, 0)),
            pl.BlockSpec((8, 128), lambda i: (0, 0)),
        ],
        out_shape=[
            jax.ShapeDtypeStruct((N_NODES, NH), jnp.float32),
            jax.ShapeDtypeStruct((8, 128), jnp.float32),
        ],
    )(x, sum_full, sq_full, mn_a, mx_a, cnt, pw, pb.reshape(1, T * F_OUT), lw,
      lb.reshape(1, NH))


def _bn_res_body(x_ref, h_ref, st_ref, g_ref, b_ref, o_ref, op_ref):
    mean = st_ref[0:1, 0:NH] * (1.0 / N_NODES)
    var = st_ref[1:2, 0:NH] * (1.0 / N_NODES) - mean * mean
    rstd = lax.rsqrt(var + 1e-5)
    hn = g_ref[...] * (h_ref[...] - mean) * rstd + b_ref[...]
    xn = (x_ref[...] + jnp.maximum(hn, 0.0)) * 0.5
    o_ref[...] = xn
    op_ref[...] = jnp.concatenate(
        [xn, jnp.zeros((xn.shape[0], DPAD - NH), jnp.float32)], axis=-1)


def _bn_res(x, h, st, g, b, bm=1000):
    return pl.pallas_call(
        _bn_res_body,
        grid=(N_NODES // bm,),
        in_specs=[
            pl.BlockSpec((bm, NH), lambda i: (i, 0)),
            pl.BlockSpec((bm, NH), lambda i: (i, 0)),
            pl.BlockSpec((8, 128), lambda i: (0, 0)),
            pl.BlockSpec((1, NH), lambda i: (0, 0)),
            pl.BlockSpec((1, NH), lambda i: (0, 0)),
        ],
        out_specs=[
            pl.BlockSpec((bm, NH), lambda i: (i, 0)),
            pl.BlockSpec((bm, DPAD), lambda i: (i, 0)),
        ],
        out_shape=[
            jax.ShapeDtypeStruct((N_NODES, NH), jnp.float32),
            jax.ShapeDtypeStruct((N_NODES, DPAD), jnp.float32),
        ],
    )(x, h, st, g.reshape(1, NH), b.reshape(1, NH))


def _head_body(a_ref, b_ref, c_ref, w1a_ref, w1b_ref, w1c_ref, b1_ref,
               w2_ref, b2_ref, o_ref):
    z = jnp.dot(a_ref[...], w1a_ref[...], preferred_element_type=jnp.float32)
    z += jnp.dot(b_ref[...], w1b_ref[...], preferred_element_type=jnp.float32)
    z += jnp.dot(c_ref[...], w1c_ref[...], preferred_element_type=jnp.float32)
    z = jnp.maximum(z + b1_ref[...], 0.0)
    y = jnp.dot(z, w2_ref[...], preferred_element_type=jnp.float32)
    o_ref[...] = jax.nn.sigmoid(y + b2_ref[...])


def _head(xs, xd, ea, w1a, w1b, w1c, b1, w2, b2, bm=1000):
    m = xs.shape[0]
    w2p = jnp.concatenate([w2, jnp.zeros((NH, 127), jnp.float32)], axis=1)
    b2p = jnp.concatenate([b2, jnp.zeros((127,), jnp.float32)]).reshape(1, 128)
    return pl.pallas_call(
        _head_body,
        grid=(m // bm,),
        in_specs=[
            pl.BlockSpec((bm, DPAD), lambda i: (i, 0)),
            pl.BlockSpec((bm, DPAD), lambda i: (i, 0)),
            pl.BlockSpec((bm, NH), lambda i: (i, 0)),
            pl.BlockSpec((DPAD, NH), lambda i: (0, 0)),
            pl.BlockSpec((DPAD, NH), lambda i: (0, 0)),
            pl.BlockSpec((NH, NH), lambda i: (0, 0)),
            pl.BlockSpec((1, NH), lambda i: (0, 0)),
            pl.BlockSpec((NH, 128), lambda i: (0, 0)),
            pl.BlockSpec((1, 128), lambda i: (0, 0)),
        ],
        out_specs=pl.BlockSpec((bm, 128), lambda i: (i, 0)),
        out_shape=jax.ShapeDtypeStruct((m, 128), jnp.float32),
    )(xs, xd, ea, w1a, w1b, w1c, b1.reshape(1, NH), w2p, b2p)


def _padw(w):
    return jnp.concatenate([w, jnp.zeros((DPAD - NH, w.shape[1]), w.dtype)])


def kernel(x, edge_index, edge_attr, pos_edge_index, pos_edge_attr,
           neg_edge_index, neg_edge_attr, params):
    e = edge_index.shape[1]
    e_lp = pos_edge_index.shape[1]
    src, dst = edge_index[0], edge_index[1]

    x0 = _mm(x, params["node_emb"]["w"], params["node_emb"]["b"])
    eattr = _mm(edge_attr, params["edge_emb"]["w"], params["edge_emb"]["b"])
    pos_ea = _mm(pos_edge_attr, params["edge_emb"]["w"],
                 params["edge_emb"]["b"])
    neg_ea = _mm(neg_edge_attr, params["edge_emb"]["w"],
                 params["edge_emb"]["b"])

    idx_layer = _pad_idx(jnp.concatenate([dst, src]), NW * GW)
    idx_heads = _pad_idx(
        jnp.concatenate([pos_edge_index[0], pos_edge_index[1],
                         neg_edge_index[0], neg_edge_index[1]]), NW * GW)

    eb = 2000
    dst3 = dst.reshape(e // eb, 1, eb)
    half_fill = (jnp.arange(e, dtype=jnp.int32) % DUMP) + HALF
    idx_seg = jnp.concatenate([
        jnp.where(dst < HALF, dst, half_fill),
        jnp.where(dst >= HALF, dst - HALF, half_fill),
    ])

    xt = x0
    xt_pad = jnp.concatenate(
        [xt, jnp.zeros((N_NODES, DPAD - NH), jnp.float32)], axis=-1)
    g = None
    cnt = None
    for lp in params["layers"]:
        if g is None:
            g = _sc_gather(xt_pad, idx_layer)
        xd_g, xs_g = g[:e], g[e:2 * e]

        wpre = jnp.concatenate([p["w"] for p in lp["pre"]], axis=1)
        bpre = jnp.concatenate([p["b"] for p in lp["pre"]])
        wpre = jnp.pad(wpre, ((0, 0), (0, FW - T * F_IN)))
        bpre = jnp.pad(bpre, (0, FW - T * F_IN))
        wd, ws, we = wpre[:NH], wpre[NH:2 * NH], wpre[2 * NH:]
        wee = lp["edge_enc"]["w"] @ we
        bee = lp["edge_enc"]["b"] @ we + bpre
        msgs, msgs2 = _mm3_sq(xd_g, xs_g, eattr, _padw(wd), _padw(ws), wee,
                              bee)

        sums8 = _sc_segsum(msgs, msgs2, idx_seg)
        mn_a, mx_a, cnt_new = _minmax_cnt(msgs, dst3, eb, with_cnt=cnt is None)
        if cnt is None:
            cnt = cnt_new
        rows = jnp.concatenate(
            [sums8[:, 0], sums8[:, 1, :N_NODES - HALF]], axis=1)
        sum_full = jnp.concatenate([rows[0], rows[1], rows[2], rows[3]],
                                   axis=-1)
        sq_full = jnp.concatenate([rows[4], rows[5], rows[6], rows[7]],
                                  axis=-1)

        pwb = jnp.zeros((T * 1300, T * F_OUT), jnp.float32)
        for t in range(T):
            pwb = pwb.at[t * 1300:(t + 1) * 1300,
                         t * F_OUT:(t + 1) * F_OUT].set(lp["post"][t]["w"])
        pbb = jnp.concatenate([p["b"] for p in lp["post"]])

        h, st = _post(xt, sum_full, sq_full, mn_a, mx_a, cnt, pwb, pbb,
                      lp["lin"]["w"], lp["lin"]["b"])
        xt, xt_pad = _bn_res(xt, h, st, lp["bn_g"], lp["bn_b"])

        g = _sc_gather(xt_pad, idx_layer)
        xd2, xs2 = g[:e], g[e:2 * e]
        w1 = lp["emlp1"]["w"]
        z = _mm3(xs2, xd2, eattr, _padw(w1[:NH]), _padw(w1[NH:2 * NH]),
                 w1[2 * NH:], lp["emlp1"]["b"], relu=True)
        eattr = _mm_res(z, lp["emlp2"]["w"], lp["emlp2"]["b"], eattr)

    gh = _sc_gather(xt_pad, idx_heads)
    ps, pd = gh[:e_lp], gh[e_lp:2 * e_lp]
    ns, nd = gh[2 * e_lp:3 * e_lp], gh[3 * e_lp:4 * e_lp]
    d1 = params["dec1"]["w"]
    pos_out = _head(ps, pd, pos_ea, _padw(d1[:NH]), _padw(d1[NH:2 * NH]),
                    d1[2 * NH:], params["dec1"]["b"],
                    params["dec2"]["w"], params["dec2"]["b"])
    neg_out = _head(ns, nd, neg_ea, _padw(d1[:NH]), _padw(d1[NH:2 * NH]),
                    d1[2 * NH:], params["dec1"]["b"],
                    params["dec2"]["w"], params["dec2"]["b"])
    return pos_out[:, 0], neg_out[:, 0]
